# baseline (speedup 1.0000x reference)
import functools

import jax
import jax.numpy as jnp
from jax import lax
from jax.experimental import pallas as pl
from jax.experimental.pallas import tpu as pltpu
from jax.experimental.pallas import tpu_sc as plsc

_NC = 2
_NS = 16
_NW = _NC * _NS
_L = 16
_PACK = 8


def kernel(xs, input_table, gt_table):
    B = xs.shape[0]
    V, D = input_table.shape
    assert B % (8 * _NW) == 0 and D == _L and V % _PACK == 0
    b_per_w = B // _NW
    n_chunks = b_per_w // _L
    flat_per_w = b_per_w * D

    in_packed = input_table.reshape(V // _PACK, _PACK * D)
    gt_packed = gt_table.reshape(V // _PACK, _PACK * D)

    mesh = plsc.VectorSubcoreMesh(core_axis_name="c", subcore_axis_name="s")

    @functools.partial(
        pl.kernel,
        mesh=mesh,
        compiler_params=pltpu.CompilerParams(needs_layout_passes=False),
        out_type=(
            jax.ShapeDtypeStruct((B * D,), jnp.float32),
            jax.ShapeDtypeStruct((B * D,), jnp.float32),
        ),
        scratch_types=[
            pltpu.VMEM((b_per_w,), jnp.float32),
            pltpu.VMEM((b_per_w,), jnp.int32),
            pltpu.VMEM((b_per_w,), jnp.int32),
            pltpu.VMEM((b_per_w, _PACK * D), jnp.float32),
            pltpu.VMEM((flat_per_w,), jnp.float32),
            pltpu.VMEM((flat_per_w,), jnp.float32),
            pltpu.SemaphoreType.DMA,
        ],
    )
    def sc_kernel(xs_hbm, in_hbm, gt_hbm, out_in_hbm, out_gt_hbm,
                  xs_v, idx_v, pidx_v, rows_v, outa_v, outb_v, sem):
        wid = lax.axis_index("s") * _NC + lax.axis_index("c")
        base = wid * b_per_w

        pltpu.sync_copy(xs_hbm.at[pl.ds(base, b_per_w)], xs_v)

        scale = jnp.float32(V)
        hi = jnp.float32(V - 1)

        def idx_body(i, carry):
            v = xs_v[pl.ds(i * _L, _L)]
            scaled = v * scale
            clipped = jnp.minimum(jnp.maximum(scaled, jnp.float32(0.0)), hi)
            iv = clipped.astype(jnp.int32)
            idx_v[pl.ds(i * _L, _L)] = iv
            pidx_v[pl.ds(i * _L, _L)] = lax.shift_right_logical(iv, 3)
            return carry

        lax.fori_loop(0, n_chunks, idx_body, 0)

        iota16 = lax.iota(jnp.int32, _L)

        def extract(out_v):
            def cbody(c, carry):
                iv = idx_v[pl.ds(c * _L, _L)]
                m16 = (iv & 7) * D
                rloc = c * _L + iota16
                fb = c * (_L * D) + iota16 * D
                for d in range(D):
                    vals = plsc.load_gather(rows_v, [rloc, m16 + d])
                    plsc.store_scatter(out_v, [fb + d], vals)
                return carry
            lax.fori_loop(0, n_chunks, cbody, 0)

        pltpu.async_copy(in_hbm.at[pidx_v], rows_v, sem).wait()
        extract(outa_v)
        pltpu.async_copy(gt_hbm.at[pidx_v], rows_v, sem).wait()
        extract(outb_v)

        pltpu.sync_copy(outa_v, out_in_hbm.at[pl.ds(base * D, flat_per_w)])
        pltpu.sync_copy(outb_v, out_gt_hbm.at[pl.ds(base * D, flat_per_w)])

    out_in, out_gt = sc_kernel(xs, in_packed, gt_packed)
    return out_in.reshape(B, D), out_gt.reshape(B, D)
